# baseline (speedup 1.0000x reference)
import jax
import jax.numpy as jnp
from jax import lax
from jax.experimental import pallas as pl
from jax.experimental.pallas import tpu as pltpu
from jax.experimental.pallas import tpu_sc as plsc

NUM_WORKERS = 32
BATCH = 16384
EMBED_DIM = 64
B_PER_W = BATCH // NUM_WORKERS
CHUNK = 128
N_CHUNKS = B_PER_W // CHUNK


def _fast_rsqrt(x):
    i = lax.bitcast_convert_type(x, jnp.int32)
    i = jnp.int32(0x5F3759DF) - (i >> 1)
    y = lax.bitcast_convert_type(i, jnp.float32)
    xh = x * jnp.float32(0.5)
    th = jnp.float32(1.5)
    y = y * (th - xh * y * y)
    y = y * (th - xh * y * y)
    y = y * (th - xh * y * y)
    return y


def _sc_body(s_hbm, o_hbm, r_hbm, e_tab, r_tab, out_hbm,
             idx_s, idx_o, idx_r, rows_s, rows_o, rows_r, out_v, sem):
    wid = lax.axis_index("s") * 2 + lax.axis_index("c")
    row0 = wid * N_CHUNKS
    base = wid * B_PER_W

    pltpu.sync_copy(s_hbm.at[pl.ds(row0, N_CHUNKS)], idx_s)
    pltpu.sync_copy(o_hbm.at[pl.ds(row0, N_CHUNKS)], idx_o)
    pltpu.sync_copy(r_hbm.at[pl.ds(row0, N_CHUNKS)], idx_r)

    copies = []
    for j in range(N_CHUNKS):
        dst = pl.ds(j * CHUNK, CHUNK)
        copies.append(pltpu.async_copy(e_tab.at[idx_s.at[j]], rows_s.at[dst], sem))
        copies.append(pltpu.async_copy(e_tab.at[idx_o.at[j]], rows_o.at[dst], sem))
        copies.append(pltpu.async_copy(r_tab.at[idx_r.at[j]], rows_r.at[dst], sem))
    for c in copies:
        c.wait()

    lane = lax.iota(jnp.int32, 16)

    def group(g, carry):
        vals = jnp.zeros((16,), jnp.float32)
        for l in range(16):
            e = g * 16 + l
            a = [rows_s[e, pl.ds(16 * k, 16)] for k in range(4)]
            b = [rows_r[e, pl.ds(16 * k, 16)] for k in range(4)]
            c = [rows_o[e, pl.ds(16 * k, 16)] for k in range(4)]
            ss = a[0] * a[0] + a[1] * a[1] + a[2] * a[2] + a[3] * a[3]
            sr = b[0] * b[0] + b[1] * b[1] + b[2] * b[2] + b[3] * b[3]
            so = c[0] * c[0] + c[1] * c[1] + c[2] * c[2] + c[3] * c[3]
            inv_s = _fast_rsqrt(jnp.sum(ss))
            inv_r = _fast_rsqrt(jnp.sum(sr))
            inv_o = _fast_rsqrt(jnp.sum(so))
            acc = jnp.abs(a[0] * inv_s + b[0] * inv_r - c[0] * inv_o)
            for k in range(1, 4):
                acc = acc + jnp.abs(a[k] * inv_s + b[k] * inv_r - c[k] * inv_o)
            vals = jnp.where(lane == l, jnp.sum(acc), vals)
        out_v[pl.ds(g * 16, 16)] = vals
        return carry

    lax.fori_loop(0, B_PER_W // 16, group, jnp.int32(0))

    pltpu.sync_copy(out_v, out_hbm.at[pl.ds(base, B_PER_W)])


@jax.jit
def kernel(s, r, o, e_embeddings, r_embeddings):
    s2 = s.astype(jnp.int32).reshape(BATCH // CHUNK, CHUNK)
    o2 = o.astype(jnp.int32).reshape(BATCH // CHUNK, CHUNK)
    r2 = r.astype(jnp.int32).reshape(BATCH // CHUNK, CHUNK)

    mesh = plsc.VectorSubcoreMesh(core_axis_name="c", subcore_axis_name="s")
    run = pl.kernel(
        _sc_body,
        out_type=jax.ShapeDtypeStruct((BATCH,), jnp.float32),
        mesh=mesh,
        compiler_params=pltpu.CompilerParams(
            needs_layout_passes=False, use_tc_tiling_on_sc=False),
        scratch_types=[
            pltpu.VMEM((N_CHUNKS, CHUNK), jnp.int32),
            pltpu.VMEM((N_CHUNKS, CHUNK), jnp.int32),
            pltpu.VMEM((N_CHUNKS, CHUNK), jnp.int32),
            pltpu.VMEM((B_PER_W, EMBED_DIM), jnp.float32),
            pltpu.VMEM((B_PER_W, EMBED_DIM), jnp.float32),
            pltpu.VMEM((B_PER_W, EMBED_DIM), jnp.float32),
            pltpu.VMEM((B_PER_W,), jnp.float32),
            pltpu.SemaphoreType.DMA,
        ],
    )
    return run(s2, o2, r2, e_embeddings, r_embeddings)
